# baseline (speedup 1.0000x reference)
import functools

import jax
import jax.numpy as jnp
from jax import lax
from jax.experimental import pallas as pl
from jax.experimental.pallas import tpu as pltpu
from jax.experimental.pallas import tpu_sc as plsc

H = 128
SEQ = 200
BATCH = 1024
NSPK = 10
N = BATCH * SEQ
NC, NS = 2, 16
NW = NC * NS
PER_W = N // NW
C = 128
NCHUNK = PER_W // C
NCH_P = NCHUNK + 2


def _tc_combo(speaker_ids2d, pos_rows, speaker_emb, fc_W, fc_b2d):
    def body(spk_ref, pos_ref, semb_ref, w_ref, b_ref, combo_ref, bidx_ref):
        x = jnp.dot(semb_ref[...], w_ref[...],
                    preferred_element_type=jnp.float32) + b_ref[...]
        feat = jnp.maximum(x, 0.0) + jnp.log1p(jnp.exp(-jnp.abs(x)))
        combo_ref[...] = feat[:, None, :] + pos_ref[...][None, :, :]
        bidx_ref[...] = spk_ref[...] * SEQ + lax.broadcasted_iota(
            jnp.int32, (BATCH, SEQ), 1)

    return pl.pallas_call(
        body,
        out_shape=(
            jax.ShapeDtypeStruct((NSPK, SEQ, H), jnp.float32),
            jax.ShapeDtypeStruct((BATCH, SEQ), jnp.int32),
        ),
    )(speaker_ids2d, pos_rows, speaker_emb, fc_W, fc_b2d)


def _sc_gather(char_emb, mixp, combo2d):
    mesh = plsc.VectorSubcoreMesh(core_axis_name="c", subcore_axis_name="s")

    @functools.partial(
        pl.kernel,
        out_type=jax.ShapeDtypeStruct((N, H), jnp.float32),
        mesh=mesh,
        scratch_types=[
            pltpu.VMEM((NCH_P, 2, C), jnp.int32),
            pltpu.VMEM((C, H), jnp.float32),
            pltpu.VMEM((C, H), jnp.float32),
            pltpu.VMEM((C, H), jnp.float32),
            pltpu.VMEM((C, H), jnp.float32),
            pltpu.VMEM((C, H), jnp.float32),
            pltpu.VMEM((C, H), jnp.float32),
            pltpu.SemaphoreType.DMA((3,)),
            pltpu.SemaphoreType.DMA((3,)),
            pltpu.SemaphoreType.DMA((3,)),
        ],
    )
    def k(char_hbm, mix_hbm, combo_hbm, out_hbm,
          ibuf, r0, r1, r2, c0, c1, c2, gsem, csem, osem):
        rows = (r0, r1, r2)
        crows = (c0, c1, c2)
        wid = lax.axis_index("s") * NC + lax.axis_index("c")
        base_w = wid * PER_W
        pltpu.sync_copy(mix_hbm.at[pl.ds(wid * NCH_P, NCH_P)], ibuf)

        def fire_g(c, p):
            pltpu.async_copy(char_hbm.at[ibuf.at[c, 0]], rows[p], gsem.at[p])
            pltpu.async_copy(combo_hbm.at[ibuf.at[c, 1]], crows[p], csem.at[p])

        def wait_g(c, p):
            pltpu.make_async_copy(char_hbm.at[ibuf.at[c, 0]], rows[p],
                                  gsem.at[p]).wait()
            pltpu.make_async_copy(combo_hbm.at[ibuf.at[c, 1]], crows[p],
                                  csem.at[p]).wait()

        def out_slice(c):
            return out_hbm.at[pl.ds(base_w + c * C, C)]

        def fire_w(c, p):
            pltpu.async_copy(rows[p], out_slice(c), osem.at[p])

        def wait_w(c, p):
            pltpu.make_async_copy(rows[p], out_slice(c), osem.at[p]).wait()

        def valu_add(p):
            rp, cp = rows[p], crows[p]

            def row4(i, carry):
                r = i * 4
                for rr in range(4):
                    for j in range(H // 16):
                        plsc.addupdate(rp.at[r + rr, pl.ds(j * 16, 16)],
                                       cp[r + rr, pl.ds(j * 16, 16)])
                return carry

            lax.fori_loop(0, C // 4, row4, 0)

        fire_g(0, 0)
        fire_g(1, 1)
        wait_g(0, 0)
        valu_add(0)
        fire_w(0, 0)
        fire_g(2, 2)
        wait_g(1, 1)
        valu_add(1)
        fire_w(1, 1)
        wait_w(0, 0)
        fire_g(3, 0)

        def steady(i, carry):
            for k3 in range(3):
                c = 3 * i + 2 + k3
                p = (2 + k3) % 3
                wait_g(c, p)
                valu_add(p)
                fire_w(c, p)
                wait_w(c - 1, (p + 2) % 3)
                fire_g(c + 2, (p + 2) % 3)
            return carry

        lax.fori_loop(0, (NCHUNK - 2) // 3, steady, 0)

        wait_g(NCHUNK, NCHUNK % 3)
        wait_g(NCHUNK + 1, (NCHUNK + 1) % 3)
        wait_w(NCHUNK - 1, (NCHUNK - 1) % 3)

    return k(char_emb, mixp, combo2d)


def kernel(input_ids, speaker_ids, char_emb, pos_table, speaker_emb, fc_W, fc_b):
    pos_rows = lax.slice_in_dim(pos_table, 1, SEQ + 1, axis=0)
    combo3, bidx = _tc_combo(speaker_ids[:, None].astype(jnp.int32), pos_rows,
                             speaker_emb, fc_W, fc_b[None, :])
    ids2d = input_ids.reshape(N // C, C)
    bidx2d = bidx.reshape(N // C, C)
    mix = jnp.stack([ids2d, bidx2d], axis=1).reshape(NW, NCHUNK, 2, C)
    mixp = jnp.concatenate(
        [mix, jnp.zeros((NW, NCH_P - NCHUNK, 2, C), jnp.int32)],
        axis=1).reshape(NW * NCH_P, 2, C)
    out = _sc_gather(char_emb, mixp, combo3.reshape(NSPK * SEQ, H))
    return out.reshape(BATCH, SEQ, H)


# speedup vs baseline: 2.3052x; 2.3052x over previous
import functools

import jax
import jax.numpy as jnp
from jax import lax
from jax.experimental import pallas as pl
from jax.experimental.pallas import tpu as pltpu
from jax.experimental.pallas import tpu_sc as plsc

H = 128
SEQ = 200
BATCH = 1024
NSPK = 10
N = BATCH * SEQ
NC, NS = 2, 16
NW = NC * NS
PER_W = N // NW
C = 128
NCHUNK = PER_W // C
NCH_P = NCHUNK + 2


def _tc_combo(speaker_ids2d, pos_rows, speaker_emb, fc_W, fc_b2d):
    def body(spk_ref, pos_ref, semb_ref, w_ref, b_ref, combo_ref, bidx_ref):
        x = jnp.dot(semb_ref[...], w_ref[...],
                    preferred_element_type=jnp.float32) + b_ref[...]
        feat = jnp.maximum(x, 0.0) + jnp.log1p(jnp.exp(-jnp.abs(x)))
        combo_ref[...] = feat[:, None, :] + pos_ref[...][None, :, :]
        bidx_ref[...] = spk_ref[...] * SEQ + lax.broadcasted_iota(
            jnp.int32, (BATCH, SEQ), 1)

    return pl.pallas_call(
        body,
        out_shape=(
            jax.ShapeDtypeStruct((NSPK, SEQ, H), jnp.float32),
            jax.ShapeDtypeStruct((BATCH, SEQ), jnp.int32),
        ),
    )(speaker_ids2d, pos_rows, speaker_emb, fc_W, fc_b2d)


def _sc_gather(char_emb, mixp, combo2d):
    mesh = plsc.VectorSubcoreMesh(core_axis_name="c", subcore_axis_name="s")

    @functools.partial(
        pl.kernel,
        out_type=jax.ShapeDtypeStruct((N, H), jnp.float32),
        mesh=mesh,
        scratch_types=[
            pltpu.VMEM((NCH_P, 2, C), jnp.int32),
            pltpu.VMEM((C, H), jnp.float32),
            pltpu.VMEM((C, H), jnp.float32),
            pltpu.VMEM((C, H), jnp.float32),
            pltpu.VMEM((C, H), jnp.float32),
            pltpu.VMEM((C, H), jnp.float32),
            pltpu.VMEM((C, H), jnp.float32),
            pltpu.SemaphoreType.DMA((3,)),
            pltpu.SemaphoreType.DMA((3,)),
            pltpu.SemaphoreType.DMA((3,)),
        ],
    )
    def k(char_hbm, mix_hbm, combo_hbm, out_hbm,
          ibuf, r0, r1, r2, c0, c1, c2, gsem, csem, osem):
        rows = (r0, r1, r2)
        crows = (c0, c1, c2)
        wid = lax.axis_index("s") * NC + lax.axis_index("c")
        base_w = wid * PER_W
        pltpu.sync_copy(mix_hbm.at[pl.ds(wid * NCH_P, NCH_P)], ibuf)

        def fire_g(c, p):
            pltpu.async_copy(char_hbm.at[ibuf.at[c, 0]], rows[p], gsem.at[p])
            pltpu.async_copy(combo_hbm.at[ibuf.at[c, 1]], crows[p], csem.at[p])

        def wait_g(c, p):
            pltpu.make_async_copy(char_hbm.at[ibuf.at[c, 0]], rows[p],
                                  gsem.at[p]).wait()
            pltpu.make_async_copy(combo_hbm.at[ibuf.at[c, 1]], crows[p],
                                  csem.at[p]).wait()

        def out_slice(c):
            return out_hbm.at[pl.ds(base_w + c * C, C)]

        def fire_w(c, p):
            pltpu.async_copy(rows[p], out_slice(c), osem.at[p])

        def wait_w(c, p):
            pltpu.make_async_copy(rows[p], out_slice(c), osem.at[p]).wait()

        def valu_add(p):
            rp, cp = rows[p], crows[p]

            def row4(i, carry):
                r = i * 4
                for rr in range(4):
                    for j in range(H // 16):
                        plsc.addupdate(rp.at[r + rr, pl.ds(j * 16, 16)],
                                       cp[r + rr, pl.ds(j * 16, 16)])
                return carry

            lax.fori_loop(0, C // 4, row4, 0)

        def chunk(c, carry):
            fire_g(c, 0)
            wait_g(c, 0)
            valu_add(0)
            fire_w(c, 0)
            wait_w(c, 0)
            return carry

        lax.fori_loop(0, NCHUNK, chunk, 0)

    return k(char_emb, mixp, combo2d)


def kernel(input_ids, speaker_ids, char_emb, pos_table, speaker_emb, fc_W, fc_b):
    pos_rows = lax.slice_in_dim(pos_table, 1, SEQ + 1, axis=0)
    combo3, bidx = _tc_combo(speaker_ids[:, None].astype(jnp.int32), pos_rows,
                             speaker_emb, fc_W, fc_b[None, :])
    ids2d = input_ids.reshape(N // C, C)
    bidx2d = bidx.reshape(N // C, C)
    mix = jnp.stack([ids2d, bidx2d], axis=1).reshape(NW, NCHUNK, 2, C)
    mixp = jnp.concatenate(
        [mix, jnp.zeros((NW, NCH_P - NCHUNK, 2, C), jnp.int32)],
        axis=1).reshape(NW * NCH_P, 2, C)
    out = _sc_gather(char_emb, mixp, combo3.reshape(NSPK * SEQ, H))
    return out.reshape(BATCH, SEQ, H)


# speedup vs baseline: 2.5960x; 1.1262x over previous
import functools

import jax
import jax.numpy as jnp
from jax import lax
from jax.experimental import pallas as pl
from jax.experimental.pallas import tpu as pltpu
from jax.experimental.pallas import tpu_sc as plsc

H = 128
SEQ = 200
BATCH = 1024
NSPK = 10
N = BATCH * SEQ
NC, NS = 2, 16
NW = NC * NS
PER_W = N // NW
C = 128
NCHUNK = PER_W // C
NCH_P = NCHUNK + 2


def _tc_combo(speaker_ids2d, pos_rows, speaker_emb, fc_W, fc_b2d):
    def body(spk_ref, pos_ref, semb_ref, w_ref, b_ref, combo_ref, bidx_ref):
        x = jnp.dot(semb_ref[...], w_ref[...],
                    preferred_element_type=jnp.float32) + b_ref[...]
        feat = jnp.maximum(x, 0.0) + jnp.log1p(jnp.exp(-jnp.abs(x)))
        combo_ref[...] = feat[:, None, :] + pos_ref[...][None, :, :]
        bidx_ref[...] = spk_ref[...] * SEQ + lax.broadcasted_iota(
            jnp.int32, (BATCH, SEQ), 1)

    return pl.pallas_call(
        body,
        out_shape=(
            jax.ShapeDtypeStruct((NSPK, SEQ, H), jnp.float32),
            jax.ShapeDtypeStruct((BATCH, SEQ), jnp.int32),
        ),
    )(speaker_ids2d, pos_rows, speaker_emb, fc_W, fc_b2d)


def _sc_gather(char_emb, mixp, combo2d):
    mesh = plsc.VectorSubcoreMesh(core_axis_name="c", subcore_axis_name="s")

    @functools.partial(
        pl.kernel,
        out_type=jax.ShapeDtypeStruct((N, H), jnp.float32),
        mesh=mesh,
        scratch_types=[
            pltpu.VMEM((NCH_P, 2, C), jnp.int32),
            pltpu.VMEM((C, H), jnp.float32),
            pltpu.VMEM((C, H), jnp.float32),
            pltpu.VMEM((C, H), jnp.float32),
            pltpu.VMEM((C, H), jnp.float32),
            pltpu.VMEM((C, H), jnp.float32),
            pltpu.VMEM((C, H), jnp.float32),
            pltpu.VMEM_SHARED((NSPK * SEQ, H), jnp.float32),
            pltpu.SemaphoreType.DMA((3,)),
            pltpu.SemaphoreType.DMA((3,)),
            pltpu.SemaphoreType.DMA((3,)),
        ],
    )
    def k(char_hbm, mix_hbm, combo_hbm, out_hbm,
          ibuf, r0, r1, r2, c0, c1, c2, combo_sh, gsem, csem, osem):
        rows = (r0, r1, r2)
        crows = (c0, c1, c2)
        wid = lax.axis_index("s") * NC + lax.axis_index("c")
        base_w = wid * PER_W

        @pl.when(lax.axis_index("s") == 0)
        def _stage_combo():
            pltpu.sync_copy(combo_hbm, combo_sh)

        pltpu.sync_copy(mix_hbm.at[pl.ds(wid * NCH_P, NCH_P)], ibuf)
        plsc.subcore_barrier()

        def fire_g(c, p):
            pltpu.async_copy(char_hbm.at[ibuf.at[c, 0]], rows[p], gsem.at[p])
            pltpu.async_copy(combo_sh.at[ibuf.at[c, 1]], crows[p], csem.at[p])

        def wait_g(c, p):
            pltpu.make_async_copy(char_hbm.at[ibuf.at[c, 0]], rows[p],
                                  gsem.at[p]).wait()
            pltpu.make_async_copy(combo_sh.at[ibuf.at[c, 1]], crows[p],
                                  csem.at[p]).wait()

        def out_slice(c):
            return out_hbm.at[pl.ds(base_w + c * C, C)]

        def fire_w(c, p):
            pltpu.async_copy(rows[p], out_slice(c), osem.at[p])

        def wait_w(c, p):
            pltpu.make_async_copy(rows[p], out_slice(c), osem.at[p]).wait()

        def valu_add(p):
            rp, cp = rows[p], crows[p]

            def row4(i, carry):
                r = i * 4
                for rr in range(4):
                    for j in range(H // 16):
                        plsc.addupdate(rp.at[r + rr, pl.ds(j * 16, 16)],
                                       cp[r + rr, pl.ds(j * 16, 16)])
                return carry

            lax.fori_loop(0, C // 4, row4, 0)

        def chunk(c, carry):
            fire_g(c, 0)
            wait_g(c, 0)
            valu_add(0)
            fire_w(c, 0)
            wait_w(c, 0)
            return carry

        lax.fori_loop(0, NCHUNK, chunk, 0)

    return k(char_emb, mixp, combo2d)


def kernel(input_ids, speaker_ids, char_emb, pos_table, speaker_emb, fc_W, fc_b):
    pos_rows = lax.slice_in_dim(pos_table, 1, SEQ + 1, axis=0)
    combo3, bidx = _tc_combo(speaker_ids[:, None].astype(jnp.int32), pos_rows,
                             speaker_emb, fc_W, fc_b[None, :])
    ids2d = input_ids.reshape(N // C, C)
    bidx2d = bidx.reshape(N // C, C)
    mix = jnp.stack([ids2d, bidx2d], axis=1).reshape(NW, NCHUNK, 2, C)
    mixp = jnp.concatenate(
        [mix, jnp.zeros((NW, NCH_P - NCHUNK, 2, C), jnp.int32)],
        axis=1).reshape(NW * NCH_P, 2, C)
    out = _sc_gather(char_emb, mixp, combo3.reshape(NSPK * SEQ, H))
    return out.reshape(BATCH, SEQ, H)
